# baseline (speedup 1.0000x reference)
import jax
import jax.numpy as jnp
from jax.experimental import pallas as pl
from jax.experimental.pallas import tpu as pltpu

_NUM_EXPERTS = 8
_RANK = 64
_TILE = 128
_CHUNK = 512


def _moe_body(x_ref, wg_ref, a_ref, b_ref, out_ref, imp_ref, load_ref,
              a_sc, b_sc):
    @pl.when(pl.program_id(0) == 0)
    def _prep():
        imp_ref[...] = jnp.zeros_like(imp_ref)
        load_ref[...] = jnp.zeros_like(load_ref)
        a_sc[...] = jnp.transpose(a_ref[...]).astype(jnp.bfloat16)
        half = b_ref.shape[1]
        n_chunks = half // _CHUNK

        for e in range(_NUM_EXPERTS):
            def _chunk(c, _, e=e):
                slab = b_ref[e, pl.ds(c * _CHUNK, _CHUNK), :]
                lt = jnp.transpose(slab[:, :_RANK])
                rt = jnp.transpose(slab[:, _RANK:])
                inter = jnp.stack([lt, rt], axis=-1).reshape(_RANK, -1)
                b_sc[pl.ds(e * _RANK, _RANK), pl.ds(c * 2 * _CHUNK, 2 * _CHUNK)] = (
                    inter.astype(jnp.bfloat16))
                return 0
            jax.lax.fori_loop(0, n_chunks, _chunk, 0, unroll=False)

    x = x_ref[...]
    logits = jnp.dot(x, wg_ref[...], preferred_element_type=jnp.float32)
    iota_e = jax.lax.broadcasted_iota(jnp.int32, logits.shape, 1)
    mx = jnp.max(logits, axis=1, keepdims=True)
    eid = jnp.min(
        jnp.where(logits >= mx, iota_e, _NUM_EXPERTS), axis=1, keepdims=True
    )

    counts = jnp.sum((iota_e == eid).astype(jnp.float32), axis=0)
    imp_ref[...] += counts[None, :]
    load_ref[...] += counts[None, :].astype(jnp.int32)

    xb = x.astype(jnp.bfloat16)
    h = jnp.dot(xb, a_sc[...], preferred_element_type=jnp.float32)
    col_e = jax.lax.broadcasted_iota(jnp.int32, h.shape, 1) // _RANK
    hg = jnp.where(col_e == eid, h, 0.0).astype(jnp.bfloat16)
    out_ref[...] = jnp.dot(hg, b_sc[...], preferred_element_type=jnp.float32)


def kernel(tokens, w_gate, A, B):
    b, s, d = tokens.shape
    e, r, _ = A.shape
    d_out = B.shape[1]
    flat = tokens.reshape(s, d)
    a2 = A.reshape(e * r, d)
    b2 = B.reshape(e, d_out // 2, 2 * r)

    n_tiles = s // _TILE
    out, imp, load = pl.pallas_call(
        _moe_body,
        grid=(n_tiles,),
        in_specs=[
            pl.BlockSpec((_TILE, d), lambda i: (i, 0)),
            pl.BlockSpec((d, e), lambda i: (0, 0)),
            pl.BlockSpec((e * r, d), lambda i: (0, 0)),
            pl.BlockSpec((e, d_out // 2, 2 * r), lambda i: (0, 0, 0)),
        ],
        out_specs=[
            pl.BlockSpec((_TILE, d_out), lambda i: (i, 0)),
            pl.BlockSpec((1, e), lambda i: (0, 0)),
            pl.BlockSpec((1, e), lambda i: (0, 0)),
        ],
        out_shape=[
            jax.ShapeDtypeStruct((s, d_out), jnp.float32),
            jax.ShapeDtypeStruct((1, e), jnp.float32),
            jax.ShapeDtypeStruct((1, e), jnp.int32),
        ],
        scratch_shapes=[
            pltpu.VMEM((d, e * r), jnp.bfloat16),
            pltpu.VMEM((e * r, d_out), jnp.bfloat16),
        ],
    )(flat, w_gate, a2, b2)
    return out.reshape(b, s, d_out), imp.reshape(e), load.reshape(e)


# speedup vs baseline: 1.0604x; 1.0604x over previous
import jax
import jax.numpy as jnp
from jax.experimental import pallas as pl
from jax.experimental.pallas import tpu as pltpu

_NUM_EXPERTS = 8
_RANK = 64
_TILE = 128
_CHUNK = 512


def _moe_body(x_ref, wg_ref, a_ref, b_ref, out_ref, imp_ref, load_ref,
              a_sc, b_sc):
    @pl.when(pl.program_id(0) == 0)
    def _prep():
        imp_ref[...] = jnp.zeros_like(imp_ref)
        load_ref[...] = jnp.zeros_like(load_ref)
        a_sc[...] = jnp.transpose(a_ref[...]).astype(jnp.bfloat16)
        half = b_ref.shape[1]
        n_chunks = half // _CHUNK
        for e in range(_NUM_EXPERTS):
            for c in range(n_chunks):
                slab = b_ref[e, c * _CHUNK:(c + 1) * _CHUNK, :]
                lt = jnp.transpose(slab[:, :_RANK])
                rt = jnp.transpose(slab[:, _RANK:])
                inter = jnp.stack([lt, rt], axis=-1).reshape(_RANK, -1)
                b_sc[e * _RANK:(e + 1) * _RANK,
                     c * 2 * _CHUNK:(c + 1) * 2 * _CHUNK] = (
                    inter.astype(jnp.bfloat16))

    x = x_ref[...]
    logits = jnp.dot(x, wg_ref[...], preferred_element_type=jnp.float32)
    iota_e = jax.lax.broadcasted_iota(jnp.int32, logits.shape, 1)
    mx = jnp.max(logits, axis=1, keepdims=True)
    eid = jnp.min(
        jnp.where(logits >= mx, iota_e, _NUM_EXPERTS), axis=1, keepdims=True
    )

    counts = jnp.sum((iota_e == eid).astype(jnp.float32), axis=0)
    imp_ref[...] += counts[None, :]
    load_ref[...] += counts[None, :].astype(jnp.int32)

    xb = x.astype(jnp.bfloat16)
    h = jnp.dot(xb, a_sc[...], preferred_element_type=jnp.float32)
    col_e = jax.lax.broadcasted_iota(jnp.int32, h.shape, 1) // _RANK
    hg = jnp.where(col_e == eid, h, 0.0).astype(jnp.bfloat16)
    out_ref[...] = jnp.dot(hg, b_sc[...], preferred_element_type=jnp.float32)


def kernel(tokens, w_gate, A, B):
    b, s, d = tokens.shape
    e, r, _ = A.shape
    d_out = B.shape[1]
    flat = tokens.reshape(s, d)
    a2 = A.reshape(e * r, d)
    b2 = B.reshape(e, d_out // 2, 2 * r)

    n_tiles = s // _TILE
    out, imp, load = pl.pallas_call(
        _moe_body,
        grid=(n_tiles,),
        in_specs=[
            pl.BlockSpec((_TILE, d), lambda i: (i, 0)),
            pl.BlockSpec((d, e), lambda i: (0, 0)),
            pl.BlockSpec((e * r, d), lambda i: (0, 0)),
            pl.BlockSpec((e, d_out // 2, 2 * r), lambda i: (0, 0, 0)),
        ],
        out_specs=[
            pl.BlockSpec((_TILE, d_out), lambda i: (i, 0)),
            pl.BlockSpec((1, e), lambda i: (0, 0)),
            pl.BlockSpec((1, e), lambda i: (0, 0)),
        ],
        out_shape=[
            jax.ShapeDtypeStruct((s, d_out), jnp.float32),
            jax.ShapeDtypeStruct((1, e), jnp.float32),
            jax.ShapeDtypeStruct((1, e), jnp.int32),
        ],
        scratch_shapes=[
            pltpu.VMEM((d, e * r), jnp.bfloat16),
            pltpu.VMEM((e * r, d_out), jnp.bfloat16),
        ],
    )(flat, w_gate, a2, b2)
    return out.reshape(b, s, d_out), imp.reshape(e), load.reshape(e)


# speedup vs baseline: 7.3649x; 6.9455x over previous
import jax
import jax.numpy as jnp
from jax.experimental import pallas as pl
from jax.experimental.pallas import tpu as pltpu

_NUM_EXPERTS = 8
_RANK = 64
_TILE = 256


def _moe_body(x_ref, wg_ref, a_ref, b_ref, out_ref, imp_ref, load_ref):
    @pl.when(pl.program_id(0) == 0)
    def _prep():
        imp_ref[...] = jnp.zeros_like(imp_ref)
        load_ref[...] = jnp.zeros_like(load_ref)

    x = x_ref[...]
    logits = jnp.dot(x, wg_ref[...], preferred_element_type=jnp.float32)
    iota_e = jax.lax.broadcasted_iota(jnp.int32, logits.shape, 1)
    mx = jnp.max(logits, axis=1, keepdims=True)
    eid = jnp.min(
        jnp.where(logits >= mx, iota_e, _NUM_EXPERTS), axis=1, keepdims=True
    )

    counts = jnp.sum((iota_e == eid).astype(jnp.float32), axis=0)
    imp_ref[...] += counts[None, :]
    load_ref[...] += counts[None, :].astype(jnp.int32)

    h = jax.lax.dot_general(
        x, a_ref[...], (((1,), (1,)), ((), ())),
        preferred_element_type=jnp.float32)
    col_e = jax.lax.broadcasted_iota(jnp.int32, h.shape, 1) // _RANK
    hg = jnp.where(col_e == eid, h, 0.0)

    acc = jnp.zeros(out_ref.shape, jnp.float32)
    for e in range(_NUM_EXPERTS):
        acc += jax.lax.dot_general(
            hg[:, e * _RANK:(e + 1) * _RANK], b_ref[e],
            (((1,), (1,)), ((), ())),
            preferred_element_type=jnp.float32)
    out_ref[...] = acc


def kernel(tokens, w_gate, A, B):
    b, s, d = tokens.shape
    e, r, _ = A.shape
    d_out = B.shape[1]
    flat = tokens.reshape(s, d)
    a2 = A.reshape(e * r, d)

    n_tiles = s // _TILE
    out, imp, load = pl.pallas_call(
        _moe_body,
        grid=(n_tiles,),
        in_specs=[
            pl.BlockSpec((_TILE, d), lambda i: (i, 0)),
            pl.BlockSpec((d, e), lambda i: (0, 0)),
            pl.BlockSpec((e * r, d), lambda i: (0, 0)),
            pl.BlockSpec((e, d_out, r), lambda i: (0, 0, 0)),
        ],
        out_specs=[
            pl.BlockSpec((_TILE, d_out), lambda i: (i, 0)),
            pl.BlockSpec((1, e), lambda i: (0, 0)),
            pl.BlockSpec((1, e), lambda i: (0, 0)),
        ],
        out_shape=[
            jax.ShapeDtypeStruct((s, d_out), jnp.float32),
            jax.ShapeDtypeStruct((1, e), jnp.float32),
            jax.ShapeDtypeStruct((1, e), jnp.int32),
        ],
    )(flat, w_gate, a2, B)
    return out.reshape(b, s, d_out), imp.reshape(e), load.reshape(e)


# speedup vs baseline: 11.7281x; 1.5924x over previous
import jax
import jax.numpy as jnp
from jax.experimental import pallas as pl
from jax.experimental.pallas import tpu as pltpu

_NUM_EXPERTS = 8
_RANK = 64
_TILE = 256


def _moe_body(x_ref, wg_ref, a_ref, b_ref, out_ref, imp_ref, load_ref,
              a_sc, m_sc):
    @pl.when(pl.program_id(0) == 0)
    def _prep():
        imp_ref[...] = jnp.zeros_like(imp_ref)
        load_ref[...] = jnp.zeros_like(load_ref)
        a_sc[...] = a_ref[...].astype(jnp.bfloat16)
        for e in range(_NUM_EXPERTS):
            m_sc[:, e * _RANK:(e + 1) * _RANK] = b_ref[e].astype(jnp.bfloat16)

    x = x_ref[...]
    logits = jnp.dot(x, wg_ref[...], preferred_element_type=jnp.float32)
    iota_e = jax.lax.broadcasted_iota(jnp.int32, logits.shape, 1)
    mx = jnp.max(logits, axis=1, keepdims=True)
    eid = jnp.min(
        jnp.where(logits >= mx, iota_e, _NUM_EXPERTS), axis=1, keepdims=True
    )

    counts = jnp.sum((iota_e == eid).astype(jnp.float32), axis=0)
    imp_ref[...] += counts[None, :]
    load_ref[...] += counts[None, :].astype(jnp.int32)

    xb = x.astype(jnp.bfloat16)
    h = jax.lax.dot_general(
        xb, a_sc[...], (((1,), (1,)), ((), ())),
        preferred_element_type=jnp.float32)
    col_e = jax.lax.broadcasted_iota(jnp.int32, h.shape, 1) // _RANK
    hg = jnp.where(col_e == eid, h, 0.0).astype(jnp.bfloat16)

    out_ref[...] = jax.lax.dot_general(
        hg, m_sc[...], (((1,), (1,)), ((), ())),
        preferred_element_type=jnp.float32)


def kernel(tokens, w_gate, A, B):
    b, s, d = tokens.shape
    e, r, _ = A.shape
    d_out = B.shape[1]
    flat = tokens.reshape(s, d)
    a2 = A.reshape(e * r, d)

    n_tiles = s // _TILE
    out, imp, load = pl.pallas_call(
        _moe_body,
        grid=(n_tiles,),
        in_specs=[
            pl.BlockSpec((_TILE, d), lambda i: (i, 0)),
            pl.BlockSpec((d, e), lambda i: (0, 0)),
            pl.BlockSpec((e * r, d), lambda i: (0, 0)),
            pl.BlockSpec((e, d_out, r), lambda i: (0, 0, 0)),
        ],
        out_specs=[
            pl.BlockSpec((_TILE, d_out), lambda i: (i, 0)),
            pl.BlockSpec((1, e), lambda i: (0, 0)),
            pl.BlockSpec((1, e), lambda i: (0, 0)),
        ],
        out_shape=[
            jax.ShapeDtypeStruct((s, d_out), jnp.float32),
            jax.ShapeDtypeStruct((1, e), jnp.float32),
            jax.ShapeDtypeStruct((1, e), jnp.int32),
        ],
        scratch_shapes=[
            pltpu.VMEM((e * r, d), jnp.bfloat16),
            pltpu.VMEM((d_out, e * r), jnp.bfloat16),
        ],
    )(flat, w_gate, a2, B)
    return out.reshape(b, s, d_out), imp.reshape(e), load.reshape(e)


# speedup vs baseline: 13.9482x; 1.1893x over previous
import jax
import jax.numpy as jnp
from jax.experimental import pallas as pl
from jax.experimental.pallas import tpu as pltpu

_NUM_EXPERTS = 8
_RANK = 64
_TILE = 256


def _moe_body(x_ref, wg_ref, a_ref, m_ref, out_ref, imp_ref, load_ref, a_sc):
    @pl.when(pl.program_id(0) == 0)
    def _prep():
        imp_ref[...] = jnp.zeros_like(imp_ref)
        load_ref[...] = jnp.zeros_like(load_ref)
        a_sc[...] = a_ref[...].astype(jnp.bfloat16)

    x = x_ref[...]
    logits = jnp.dot(x, wg_ref[...], preferred_element_type=jnp.float32)
    iota_e = jax.lax.broadcasted_iota(jnp.int32, logits.shape, 1)
    mx = jnp.max(logits, axis=1, keepdims=True)
    eid = jnp.min(
        jnp.where(logits >= mx, iota_e, _NUM_EXPERTS), axis=1, keepdims=True
    )

    counts = jnp.sum((iota_e == eid).astype(jnp.float32), axis=0)
    imp_ref[...] += counts[None, :]
    load_ref[...] += counts[None, :].astype(jnp.int32)

    xb = x.astype(jnp.bfloat16)
    h = jax.lax.dot_general(
        xb, a_sc[...], (((1,), (1,)), ((), ())),
        preferred_element_type=jnp.float32)
    col_e = jax.lax.broadcasted_iota(jnp.int32, h.shape, 1) // _RANK
    hg = jnp.where(col_e == eid, h, 0.0).astype(jnp.bfloat16)

    out_ref[...] = jax.lax.dot_general(
        hg, m_ref[...], (((1,), (1,)), ((), ())),
        preferred_element_type=jnp.float32)


def kernel(tokens, w_gate, A, B):
    b, s, d = tokens.shape
    e, r, _ = A.shape
    d_out = B.shape[1]
    flat = tokens.reshape(s, d)
    a2 = A.reshape(e * r, d)
    m = jnp.transpose(B, (1, 0, 2)).reshape(d_out, e * r).astype(jnp.bfloat16)

    n_tiles = s // _TILE
    out, imp, load = pl.pallas_call(
        _moe_body,
        grid=(n_tiles,),
        in_specs=[
            pl.BlockSpec((_TILE, d), lambda i: (i, 0)),
            pl.BlockSpec((d, e), lambda i: (0, 0)),
            pl.BlockSpec((e * r, d), lambda i: (0, 0)),
            pl.BlockSpec((d_out, e * r), lambda i: (0, 0)),
        ],
        out_specs=[
            pl.BlockSpec((_TILE, d_out), lambda i: (i, 0)),
            pl.BlockSpec((1, e), lambda i: (0, 0)),
            pl.BlockSpec((1, e), lambda i: (0, 0)),
        ],
        out_shape=[
            jax.ShapeDtypeStruct((s, d_out), jnp.float32),
            jax.ShapeDtypeStruct((1, e), jnp.float32),
            jax.ShapeDtypeStruct((1, e), jnp.int32),
        ],
        scratch_shapes=[
            pltpu.VMEM((e * r, d), jnp.bfloat16),
        ],
    )(flat, w_gate, a2, m)
    return out.reshape(b, s, d_out), imp.reshape(e), load.reshape(e)


# speedup vs baseline: 16.7513x; 1.2010x over previous
import jax
import jax.numpy as jnp
from jax.experimental import pallas as pl
from jax.experimental.pallas import tpu as pltpu

_NUM_EXPERTS = 8
_RANK = 64
_TILE = 256


def _moe_body(x_ref, wg_ref, a_ref, m_ref, out_ref, imp_ref, load_ref, a_sc):
    @pl.when(pl.program_id(0) == 0)
    def _prep():
        imp_ref[...] = jnp.zeros_like(imp_ref)
        load_ref[...] = jnp.zeros_like(load_ref)
        a_sc[...] = a_ref[...].astype(jnp.bfloat16)

    x = x_ref[...]
    logits = jnp.dot(x, wg_ref[...], preferred_element_type=jnp.float32)
    iota_e = jax.lax.broadcasted_iota(jnp.int32, logits.shape, 1)
    mx = jnp.max(logits, axis=1, keepdims=True)
    eid = jnp.min(
        jnp.where(logits >= mx, iota_e, _NUM_EXPERTS), axis=1, keepdims=True
    )

    counts = jnp.sum((iota_e == eid).astype(jnp.float32), axis=0)
    imp_ref[...] += counts[None, :]
    load_ref[...] += counts[None, :].astype(jnp.int32)

    xb = x.astype(jnp.bfloat16)
    h = jax.lax.dot_general(
        xb, a_sc[...], (((1,), (1,)), ((), ())),
        preferred_element_type=jnp.float32)
    col_e = jax.lax.broadcasted_iota(jnp.int32, h.shape, 1) // _RANK
    hg = jnp.where(col_e == eid, h, 0.0).astype(jnp.bfloat16)

    out_ref[...] = jnp.dot(
        hg, m_ref[...], preferred_element_type=jnp.float32)


def kernel(tokens, w_gate, A, B):
    b, s, d = tokens.shape
    e, r, _ = A.shape
    d_out = B.shape[1]
    flat = tokens.reshape(s, d)
    a2 = A.reshape(e * r, d)
    m = jnp.transpose(B, (0, 2, 1)).reshape(e * r, d_out).astype(jnp.bfloat16)

    n_tiles = s // _TILE
    out, imp, load = pl.pallas_call(
        _moe_body,
        grid=(n_tiles,),
        in_specs=[
            pl.BlockSpec((_TILE, d), lambda i: (i, 0)),
            pl.BlockSpec((d, e), lambda i: (0, 0)),
            pl.BlockSpec((e * r, d), lambda i: (0, 0)),
            pl.BlockSpec((e * r, d_out), lambda i: (0, 0)),
        ],
        out_specs=[
            pl.BlockSpec((_TILE, d_out), lambda i: (i, 0)),
            pl.BlockSpec((1, e), lambda i: (0, 0)),
            pl.BlockSpec((1, e), lambda i: (0, 0)),
        ],
        out_shape=[
            jax.ShapeDtypeStruct((s, d_out), jnp.float32),
            jax.ShapeDtypeStruct((1, e), jnp.float32),
            jax.ShapeDtypeStruct((1, e), jnp.int32),
        ],
        scratch_shapes=[
            pltpu.VMEM((e * r, d), jnp.bfloat16),
        ],
    )(flat, w_gate, a2, m)
    return out.reshape(b, s, d_out), imp.reshape(e), load.reshape(e)
